# speedup vs baseline: 3.7569x; 3.7569x over previous
import functools

import jax
import jax.numpy as jnp
from jax import lax
from jax.experimental import pallas as pl
from jax.experimental.pallas import tpu as pltpu
from jax.experimental.pallas import tpu_sc as plsc

N = 10000
D = 128
B = 128

_info = plsc.get_sparse_core_info()
NC = _info.num_cores
NS = _info.num_subcores
NW = NC * NS

CHUNK = (N // (NW * 8)) * 8
REM = N - NW * CHUNK
assert 0 <= REM <= 128 and REM % 8 == 0

_NG = 1
while CHUNK // _NG > 128 or CHUNK % _NG or (CHUNK // _NG) % 8:
    _NG += 1
NGROUPS = _NG
GROUP = CHUNK // NGROUPS


def _sc_segment_sum(x, batch, zeros):
    @functools.partial(
        pl.kernel,
        mesh=plsc.VectorSubcoreMesh(core_axis_name="c", subcore_axis_name="s"),
        out_type=jax.ShapeDtypeStruct((NC, B, D), jnp.float32),
        scratch_types=[
            pltpu.VMEM((CHUNK, D), jnp.float32),
            pltpu.VMEM((NGROUPS, GROUP), jnp.int32),
            pltpu.VMEM((max(REM, 8), D), jnp.float32),
            pltpu.VMEM((1, max(REM, 8)), jnp.int32),
            pltpu.VMEM_SHARED((B, D), jnp.float32),
        ],
    )
    def k(x_hbm, batch_hbm, zeros_hbm, out_hbm, xbuf, idxbuf, xrem, idxrem, acc):
        c = lax.axis_index("c")
        s = lax.axis_index("s")
        wid = s * NC + c
        base = wid * CHUNK

        @pl.when(s == 0)
        def _():
            pltpu.sync_copy(zeros_hbm, acc)

        plsc.subcore_barrier()

        pltpu.sync_copy(x_hbm.at[pl.ds(base, CHUNK)], xbuf)
        for g in range(NGROUPS):
            pltpu.sync_copy(
                batch_hbm.at[pl.ds(base + g * GROUP, GROUP)], idxbuf.at[g]
            )
        for g in range(NGROUPS):
            pltpu.sync_copy(
                xbuf.at[pl.ds(g * GROUP, GROUP)],
                acc.at[idxbuf.at[g]],
                add=True,
            )

        if REM:
            @pl.when(wid == 0)
            def _():
                pltpu.sync_copy(
                    x_hbm.at[pl.ds(NW * CHUNK, REM)], xrem.at[pl.ds(0, REM)]
                )
                pltpu.sync_copy(
                    batch_hbm.at[pl.ds(NW * CHUNK, REM)],
                    idxrem.at[0, pl.ds(0, REM)],
                )
                pltpu.sync_copy(
                    xrem.at[pl.ds(0, REM)],
                    acc.at[idxrem.at[0, pl.ds(0, REM)]],
                    add=True,
                )

        plsc.subcore_barrier()

        @pl.when(s == 0)
        def _():
            pltpu.sync_copy(acc, out_hbm.at[c])

    return k(x, batch, zeros)


NPAD = ((N + 127) // 128) * 128
NROWS = NPAD // 128


def _tc_finish(partials, batch2d, u, wt, bias):
    def body(p_ref, bat_ref, u_ref, wt_ref, b_ref, o_ref):
        sums = p_ref[0] + p_ref[1]

        seg = lax.broadcasted_iota(jnp.int32, (B, 128), 0)

        def cnt_step(r, cnt):
            row = bat_ref[pl.ds(r, 1), :]
            eq = (row == seg).astype(jnp.float32)
            return cnt + jnp.sum(eq, axis=1, keepdims=True)

        counts = lax.fori_loop(0, NROWS, cnt_step, jnp.zeros((B, 1), jnp.float32))

        mean = sums / jnp.maximum(counts, 1.0)

        out = lax.dot_general(
            u_ref[...], wt_ref[pl.ds(0, D), :],
            (((1,), (0,)), ((), ())), preferred_element_type=jnp.float32,
        )
        out = out + lax.dot_general(
            mean, wt_ref[pl.ds(D, D), :],
            (((1,), (0,)), ((), ())), preferred_element_type=jnp.float32,
        )
        out = out + b_ref[...]
        o_ref[...] = jnp.maximum(out, 0.0)

    return pl.pallas_call(
        body,
        out_shape=jax.ShapeDtypeStruct((B, D), jnp.float32),
    )(partials, batch2d, u, wt, bias)


@jax.jit
def kernel(x, edge_index, edge_attr, u, batch, W, b):
    del edge_index, edge_attr
    zeros = jnp.zeros((B, D), jnp.float32)
    batch = batch.astype(jnp.int32)
    partials = _sc_segment_sum(x, batch, zeros)
    batch2d = jnp.pad(batch, (0, NPAD - N), constant_values=B + 1).reshape(NROWS, 128)
    wt = W.T
    bias = b.reshape(1, D)
    return _tc_finish(partials, batch2d, u, wt, bias)


# speedup vs baseline: 4.1934x; 1.1162x over previous
import functools

import jax
import jax.numpy as jnp
from jax import lax
from jax.experimental import pallas as pl
from jax.experimental.pallas import tpu as pltpu
from jax.experimental.pallas import tpu_sc as plsc

N = 10000
D = 128
B = 128

_info = plsc.get_sparse_core_info()
NC = _info.num_cores
NS = _info.num_subcores
NW = NC * NS

CHUNK = (N // (NW * 8)) * 8
REM = N - NW * CHUNK
assert 0 <= REM <= 128 and REM % 8 == 0

_NG = 1
while CHUNK // _NG > 128 or CHUNK % _NG or (CHUNK // _NG) % 8:
    _NG += 1
NGROUPS = _NG
GROUP = CHUNK // NGROUPS

ROWS_PER_SUB = B // NS


def _sc_segment_sum(x, batch, zeros, ones):
    @functools.partial(
        pl.kernel,
        mesh=plsc.VectorSubcoreMesh(core_axis_name="c", subcore_axis_name="s"),
        out_type=[
            jax.ShapeDtypeStruct((NC * B, D), jnp.float32),
            jax.ShapeDtypeStruct((NC * B, D), jnp.float32),
        ],
        scratch_types=[
            pltpu.VMEM((CHUNK, D), jnp.float32),
            pltpu.VMEM((NGROUPS, GROUP), jnp.int32),
            pltpu.VMEM((GROUP, D), jnp.float32),
            pltpu.VMEM((max(REM, 8), D), jnp.float32),
            pltpu.VMEM((1, max(REM, 8)), jnp.int32),
            pltpu.VMEM_SHARED((B, D), jnp.float32),
            pltpu.VMEM_SHARED((B, D), jnp.float32),
            pltpu.SemaphoreType.DMA,
            pltpu.SemaphoreType.DMA,
            pltpu.SemaphoreType.DMA,
            pltpu.SemaphoreType.DMA,
        ],
    )
    def k(x_hbm, batch_hbm, z_hbm, ones_hbm, out_hbm, cnt_hbm,
          xbuf, idxbuf, onesbuf, xrem, idxrem, acc, cacc,
          sem0, sem1, sem2, semi):
        c = lax.axis_index("c")
        s = lax.axis_index("s")
        wid = s * NC + c
        base = wid * CHUNK

        sems = [sem0, sem1, sem2]
        assert NGROUPS == len(sems)

        xcopies = [
            pltpu.async_copy(
                x_hbm.at[pl.ds(base + g * GROUP, GROUP)],
                xbuf.at[pl.ds(g * GROUP, GROUP)],
                sems[g],
            )
            for g in range(NGROUPS)
        ]
        icopies = [
            pltpu.async_copy(
                batch_hbm.at[pl.ds(base + g * GROUP, GROUP)],
                idxbuf.at[g],
                semi,
            )
            for g in range(NGROUPS)
        ]
        ocopy = pltpu.async_copy(ones_hbm, onesbuf, semi)

        @pl.when(s == 0)
        def _():
            pltpu.sync_copy(z_hbm, acc)
            pltpu.sync_copy(z_hbm, cacc)

        plsc.subcore_barrier()

        for cp in icopies:
            cp.wait()
        ocopy.wait()
        for g in range(NGROUPS):
            xcopies[g].wait()
            pltpu.sync_copy(
                xbuf.at[pl.ds(g * GROUP, GROUP)],
                acc.at[idxbuf.at[g]],
                add=True,
            )
            pltpu.sync_copy(onesbuf, cacc.at[idxbuf.at[g]], add=True)

        if REM:
            @pl.when(wid == 0)
            def _():
                pltpu.sync_copy(
                    x_hbm.at[pl.ds(NW * CHUNK, REM)], xrem.at[pl.ds(0, REM)]
                )
                pltpu.sync_copy(
                    batch_hbm.at[pl.ds(NW * CHUNK, REM)],
                    idxrem.at[0, pl.ds(0, REM)],
                )
                pltpu.sync_copy(
                    xrem.at[pl.ds(0, REM)],
                    acc.at[idxrem.at[0, pl.ds(0, REM)]],
                    add=True,
                )
                pltpu.sync_copy(
                    onesbuf.at[pl.ds(0, REM)],
                    cacc.at[idxrem.at[0, pl.ds(0, REM)]],
                    add=True,
                )

        plsc.subcore_barrier()

        row = s * ROWS_PER_SUB
        pltpu.sync_copy(
            acc.at[pl.ds(row, ROWS_PER_SUB)],
            out_hbm.at[pl.ds(c * B + row, ROWS_PER_SUB)],
        )
        pltpu.sync_copy(
            cacc.at[pl.ds(row, ROWS_PER_SUB)],
            cnt_hbm.at[pl.ds(c * B + row, ROWS_PER_SUB)],
        )

    return k(x, batch, zeros, ones)


def _tc_finish(partials, cnt, u, w, bias):
    def body(p_ref, c_ref, u_ref, w_ref, b_ref, o_ref):
        sums = p_ref[pl.ds(0, B), :] + p_ref[pl.ds(B, B), :]
        counts = c_ref[pl.ds(0, B), pl.ds(0, 1)] + c_ref[pl.ds(B, B), pl.ds(0, 1)]
        mean = sums / jnp.maximum(counts, 1.0)
        out = lax.dot_general(
            u_ref[...], w_ref[:, pl.ds(0, D)],
            (((1,), (1,)), ((), ())), preferred_element_type=jnp.float32,
        )
        out = out + lax.dot_general(
            mean, w_ref[:, pl.ds(D, D)],
            (((1,), (1,)), ((), ())), preferred_element_type=jnp.float32,
        )
        out = out + b_ref[...]
        o_ref[...] = jnp.maximum(out, 0.0)

    return pl.pallas_call(
        body,
        out_shape=jax.ShapeDtypeStruct((B, D), jnp.float32),
    )(partials, cnt, u, w, bias)


@jax.jit
def kernel(x, edge_index, edge_attr, u, batch, W, b):
    del edge_index, edge_attr
    zeros = jnp.zeros((B, D), jnp.float32)
    ones = jnp.ones((GROUP, D), jnp.float32)
    batch = batch.astype(jnp.int32)
    partials, cnt = _sc_segment_sum(x, batch, zeros, ones)
    bias = b.reshape(1, D)
    return _tc_finish(partials, cnt, u, W, bias)
